# baseline (speedup 1.0000x reference)
import jax
import jax.numpy as jnp
import numpy as np
from jax import lax
from jax.experimental import pallas as pl
from jax.experimental.pallas import tpu as pltpu
from jax.experimental.pallas import tpu_sc as plsc

_LEVELS = ((128, 128, 4.0), (64, 64, 8.0), (32, 32, 16.0))
_N = 21504
_ROWS = _N // 128
_NT = 16
_PER = _N // _NT
_NV = _PER // 16
_MAX_OUT = 512


def _static_arrays():
    ybs, xbs, uys, uxs, scs = [], [], [], [], []
    for h, w, s in _LEVELS:
        gy, gx = np.meshgrid(np.arange(h), np.arange(w), indexing="ij")
        ybs.append((gy + 0.5).astype(np.float32).ravel())
        xbs.append((gx + 0.5).astype(np.float32).ravel())
        uys.append(np.full(h * w, h, np.float32))
        uxs.append(np.full(h * w, w, np.float32))
        scs.append(np.full(h * w, s, np.float32))
    cat = lambda parts: np.concatenate(parts).reshape(_ROWS, 128)
    return cat(ybs), cat(xbs), cat(uys), cat(uxs), cat(scs)


_YB, _XB, _UY, _UX, _SC = _static_arrays()


def _pre_body(l0, l1, rz, ry, rx, yb, xb, uy, ux, sc, cur_o, z_o, y_o, x_o):
    a = l0[...]
    b = l1[...]
    mx = jnp.maximum(a, b)
    e0 = jnp.exp(a - mx)
    e1 = jnp.exp(b - mx)
    s = e0 / (e0 + e1)
    vz = 0.5 + rz[...]
    vy = yb[...] + ry[...]
    vx = xb[...] + rx[...]
    valid = (vz > 0.0) & (vz < 1.0) & (vy > 0.0) & (vy < uy[...]) & (vx > 0.0) & (vx < ux[...])
    cur_o[...] = jnp.where(valid & (s > 0.2), s, -1.0)
    z_o[...] = vz * 5.0
    y_o[...] = vy * sc[...]
    x_o[...] = vx * sc[...]


_preprocess = pl.pallas_call(
    _pre_body,
    out_shape=tuple(jax.ShapeDtypeStruct((_ROWS, 128), jnp.float32) for _ in range(4)),
)


def _nms_body(cur_h, z_h, y_h, x_h, os_h, oz_h, oy_h, ox_h, oc_h,
              ac, az, ay, ax, stage, rows, wvec,
              os_v, oz_v, oy_v, ox_v, oc_v, pub, wsh):
    sid = lax.axis_index("s")
    base = sid * _PER
    pltpu.sync_copy(cur_h.at[pl.ds(base, _PER)], ac)
    pltpu.sync_copy(z_h.at[pl.ds(base, _PER)], az)
    pltpu.sync_copy(y_h.at[pl.ds(base, _PER)], ay)
    pltpu.sync_copy(x_h.at[pl.ds(base, _PER)], ax)

    lanes = jnp.arange(16, dtype=jnp.int32)
    lanesf = lanes.astype(jnp.float32)
    basef = (sid * _PER).astype(jnp.float32)
    neg1 = jnp.full((16,), -1.0, jnp.float32)
    zeros = jnp.zeros((16,), jnp.float32)
    zeroi = jnp.zeros((16,), jnp.int32)

    def body(k, carry):
        wz, wy, wx, sacc, zacc, yacc, xacc, cacc = carry

        def scan_body(i, c):
            bv, bif, bz, by, bx = c
            off = i * 16
            cv = ac[pl.ds(off, 16)]
            zz = az[pl.ds(off, 16)]
            yy = ay[pl.ds(off, 16)]
            xx = ax[pl.ds(off, 16)]
            dz = zz - wz
            dy = yy - wy
            dx = xx - wx
            d2 = dz * dz + dy * dy + dx * dx
            nc = jnp.where(d2 < 64.0, -1.0, cv)
            ac[pl.ds(off, 16)] = nc
            better = nc > bv
            fi = basef + off.astype(jnp.float32) + lanesf
            bv = jnp.where(better, nc, bv)
            bif = jnp.where(better, fi, bif)
            bz = jnp.where(better, zz, bz)
            by = jnp.where(better, yy, by)
            bx = jnp.where(better, xx, bx)
            return (bv, bif, bz, by, bx)

        bv, bif, bz, by, bx = lax.fori_loop(
            0, _NV, scan_body, (neg1, zeros, zeros, zeros, zeros), unroll=4)

        stage[pl.ds(0, 16)] = bv
        stage[pl.ds(16, 16)] = bif
        stage[pl.ds(32, 16)] = bz
        stage[pl.ds(48, 16)] = by
        stage[pl.ds(64, 16)] = bx
        even = lax.rem(k, 2) == 0

        @pl.when(even)
        def _pub0():
            pltpu.sync_copy(stage, pub.at[pl.ds(sid * 80, 80)])

        @pl.when(jnp.logical_not(even))
        def _pub1():
            pltpu.sync_copy(stage, pub.at[pl.ds(1280 + sid * 80, 80)])

        plsc.subcore_barrier()

        @pl.when(sid == 0)
        def _reduce():
            @pl.when(even)
            def _rd0():
                pltpu.sync_copy(pub.at[pl.ds(0, 1280)], rows)

            @pl.when(jnp.logical_not(even))
            def _rd1():
                pltpu.sync_copy(pub.at[pl.ds(1280, 1280)], rows)
            gv = rows[pl.ds(0, 16)]
            gi = rows[pl.ds(16, 16)]
            gz = rows[pl.ds(32, 16)]
            gy = rows[pl.ds(48, 16)]
            gx = rows[pl.ds(64, 16)]
            for t in range(1, _NT):
                tv = rows[pl.ds(t * 80, 16)]
                ti = rows[pl.ds(t * 80 + 16, 16)]
                tz = rows[pl.ds(t * 80 + 32, 16)]
                ty = rows[pl.ds(t * 80 + 48, 16)]
                tx = rows[pl.ds(t * 80 + 64, 16)]
                btr = tv > gv
                gv = jnp.where(btr, tv, gv)
                gi = jnp.where(btr, ti, gi)
                gz = jnp.where(btr, tz, gz)
                gy = jnp.where(btr, ty, gy)
                gx = jnp.where(btr, tx, gx)
            bm = gv[0]
            bi = gi[0]
            bzs = gz[0]
            bys = gy[0]
            bxs = gx[0]
            for l in range(1, 16):
                v = gv[l]
                i_ = gi[l]
                btr = (v > bm) | ((v == bm) & (i_ < bi))
                bm = jnp.where(btr, v, bm)
                bi = jnp.where(btr, i_, bi)
                bzs = jnp.where(btr, gz[l], bzs)
                bys = jnp.where(btr, gy[l], bys)
                bxs = jnp.where(btr, gx[l], bxs)
            wv = jnp.where(lanes == 0, bm, 0.0)
            wv = jnp.where(lanes == 1, bzs, wv)
            wv = jnp.where(lanes == 2, bys, wv)
            wv = jnp.where(lanes == 3, bxs, wv)
            stage[pl.ds(0, 16)] = wv

            @pl.when(even)
            def _ww0():
                pltpu.sync_copy(stage.at[pl.ds(0, 16)], wsh.at[pl.ds(0, 16)])

            @pl.when(jnp.logical_not(even))
            def _ww1():
                pltpu.sync_copy(stage.at[pl.ds(0, 16)], wsh.at[pl.ds(16, 16)])

        plsc.subcore_barrier()

        @pl.when(even)
        def _wr0():
            pltpu.sync_copy(wsh.at[pl.ds(0, 16)], wvec)

        @pl.when(jnp.logical_not(even))
        def _wr1():
            pltpu.sync_copy(wsh.at[pl.ds(16, 16)], wvec)
        w = wvec[...]
        nm = w[0]
        nz = w[1]
        ny = w[2]
        nx = w[3]
        valid = nm > 0.0

        lane_eq = lanes == lax.rem(k, 16)
        sacc = jnp.where(lane_eq, jnp.where(valid, nm, -1.0), sacc)
        zacc = jnp.where(lane_eq, jnp.where(valid, nz * 0.2, 0.0), zacc)
        yacc = jnp.where(lane_eq, jnp.where(valid, ny, 0.0), yacc)
        xacc = jnp.where(lane_eq, jnp.where(valid, nx, 0.0), xacc)
        cacc = jnp.where(lane_eq, jnp.where(valid, 0, -1), cacc)

        @pl.when((sid == 0) & (lax.rem(k, 16) == 15))
        def _flush():
            o = (k // 16) * 16
            os_v[pl.ds(o, 16)] = sacc
            oz_v[pl.ds(o, 16)] = zacc
            oy_v[pl.ds(o, 16)] = yacc
            ox_v[pl.ds(o, 16)] = xacc
            oc_v[pl.ds(o, 16)] = cacc

        return (nz, ny, nx, sacc, zacc, yacc, xacc, cacc)

    lax.fori_loop(0, _MAX_OUT, body,
                  (jnp.float32(1e9), jnp.float32(1e9), jnp.float32(1e9),
                   neg1, zeros, zeros, zeros, zeroi))

    @pl.when(sid == 0)
    def _out():
        pltpu.sync_copy(os_v, os_h)
        pltpu.sync_copy(oz_v, oz_h)
        pltpu.sync_copy(oy_v, oy_h)
        pltpu.sync_copy(ox_v, ox_h)
        pltpu.sync_copy(oc_v, oc_h)


_nms = pl.kernel(
    _nms_body,
    out_type=(
        jax.ShapeDtypeStruct((_MAX_OUT,), jnp.float32),
        jax.ShapeDtypeStruct((_MAX_OUT,), jnp.float32),
        jax.ShapeDtypeStruct((_MAX_OUT,), jnp.float32),
        jax.ShapeDtypeStruct((_MAX_OUT,), jnp.float32),
        jax.ShapeDtypeStruct((_MAX_OUT,), jnp.int32),
    ),
    mesh=plsc.VectorSubcoreMesh(core_axis_name="c", subcore_axis_name="s", num_cores=1),
    scratch_types=[
        pltpu.VMEM((_PER,), jnp.float32),
        pltpu.VMEM((_PER,), jnp.float32),
        pltpu.VMEM((_PER,), jnp.float32),
        pltpu.VMEM((_PER,), jnp.float32),
        pltpu.VMEM((80,), jnp.float32),
        pltpu.VMEM((_NT * 80,), jnp.float32),
        pltpu.VMEM((16,), jnp.float32),
        pltpu.VMEM((_MAX_OUT,), jnp.float32),
        pltpu.VMEM((_MAX_OUT,), jnp.float32),
        pltpu.VMEM((_MAX_OUT,), jnp.float32),
        pltpu.VMEM((_MAX_OUT,), jnp.float32),
        pltpu.VMEM((_MAX_OUT,), jnp.int32),
        pltpu.VMEM_SHARED((2 * _NT * 80,), jnp.float32),
        pltpu.VMEM_SHARED((32,), jnp.float32),
    ],
)


def kernel(cls_logits_0, regressions_0, cls_logits_1, regressions_1,
           cls_logits_2, regressions_2):
    cls = (cls_logits_0, cls_logits_1, cls_logits_2)
    reg = (regressions_0, regressions_1, regressions_2)
    cat = lambda ch, arrs: jnp.concatenate(
        [a[..., ch].reshape(-1) for a in arrs]).reshape(_ROWS, 128)
    l0 = cat(0, cls)
    l1 = cat(1, cls)
    rz = cat(0, reg)
    ry = cat(1, reg)
    rx = cat(2, reg)
    cur, z, y, x = _preprocess(l0, l1, rz, ry, rx,
                               jnp.asarray(_YB), jnp.asarray(_XB),
                               jnp.asarray(_UY), jnp.asarray(_UX),
                               jnp.asarray(_SC))
    out_s, out_z, out_y, out_x, out_c = _nms(
        cur.reshape(-1), z.reshape(-1), y.reshape(-1), x.reshape(-1))
    out_locs = jnp.stack([out_z, out_y, out_x], axis=-1)
    return out_s, out_locs, out_c


# speedup vs baseline: 1.6258x; 1.6258x over previous
import jax
import jax.numpy as jnp
import numpy as np
from jax import lax
from jax.experimental import pallas as pl
from jax.experimental.pallas import tpu as pltpu
from jax.experimental.pallas import tpu_sc as plsc

_LEVELS = ((128, 128, 4.0), (64, 64, 8.0), (32, 32, 16.0))
_N = 21504
_ROWS = _N // 128
_NT = 16
_PER = _N // _NT
_NV = _PER // 16
_MAX_OUT = 512


def _static_arrays():
    ybs, xbs, uys, uxs, scs = [], [], [], [], []
    for h, w, s in _LEVELS:
        gy, gx = np.meshgrid(np.arange(h), np.arange(w), indexing="ij")
        ybs.append((gy + 0.5).astype(np.float32).ravel())
        xbs.append((gx + 0.5).astype(np.float32).ravel())
        uys.append(np.full(h * w, h, np.float32))
        uxs.append(np.full(h * w, w, np.float32))
        scs.append(np.full(h * w, s, np.float32))
    cat = lambda parts: np.concatenate(parts).reshape(_ROWS, 128)
    return cat(ybs), cat(xbs), cat(uys), cat(uxs), cat(scs)


_YB, _XB, _UY, _UX, _SC = _static_arrays()


def _pre_body(l0, l1, rz, ry, rx, yb, xb, uy, ux, sc, cur_o, z_o, y_o, x_o):
    a = l0[...]
    b = l1[...]
    mx = jnp.maximum(a, b)
    e0 = jnp.exp(a - mx)
    e1 = jnp.exp(b - mx)
    s = e0 / (e0 + e1)
    vz = 0.5 + rz[...]
    vy = yb[...] + ry[...]
    vx = xb[...] + rx[...]
    valid = (vz > 0.0) & (vz < 1.0) & (vy > 0.0) & (vy < uy[...]) & (vx > 0.0) & (vx < ux[...])
    cur_o[...] = jnp.where(valid & (s > 0.2), s, -1.0)
    z_o[...] = vz * 5.0
    y_o[...] = vy * sc[...]
    x_o[...] = vx * sc[...]


_preprocess = pl.pallas_call(
    _pre_body,
    out_shape=tuple(jax.ShapeDtypeStruct((_ROWS, 128), jnp.float32) for _ in range(4)),
)


def _nms_body(cur_h, z_h, y_h, x_h, os_h, oz_h, oy_h, ox_h, oc_h,
              ac, az, ay, ax, stage, rows,
              os_v, oz_v, oy_v, ox_v, oc_v, pub):
    sid = lax.axis_index("s")
    base = sid * _PER
    pltpu.sync_copy(cur_h.at[pl.ds(base, _PER)], ac)
    pltpu.sync_copy(z_h.at[pl.ds(base, _PER)], az)
    pltpu.sync_copy(y_h.at[pl.ds(base, _PER)], ay)
    pltpu.sync_copy(x_h.at[pl.ds(base, _PER)], ax)

    lanes = jnp.arange(16, dtype=jnp.int32)
    lanesf = lanes.astype(jnp.float32)
    basef = (sid * _PER).astype(jnp.float32)
    neg1 = jnp.full((16,), -1.0, jnp.float32)
    zeros = jnp.zeros((16,), jnp.float32)
    zeroi = jnp.zeros((16,), jnp.int32)

    def body(k, carry):
        wz, wy, wx, sacc, zacc, yacc, xacc, cacc = carry

        def scan_body(i, c):
            bv, bif, bz, by, bx = c
            off = i * 16
            cv = ac[pl.ds(off, 16)]
            zz = az[pl.ds(off, 16)]
            yy = ay[pl.ds(off, 16)]
            xx = ax[pl.ds(off, 16)]
            dz = zz - wz
            dy = yy - wy
            dx = xx - wx
            d2 = dz * dz + dy * dy + dx * dx
            nc = jnp.where(d2 < 64.0, -1.0, cv)
            ac[pl.ds(off, 16)] = nc
            better = nc > bv
            fi = basef + off.astype(jnp.float32) + lanesf
            bv = jnp.where(better, nc, bv)
            bif = jnp.where(better, fi, bif)
            bz = jnp.where(better, zz, bz)
            by = jnp.where(better, yy, by)
            bx = jnp.where(better, xx, bx)
            return (bv, bif, bz, by, bx)

        bv, bif, bz, by, bx = lax.fori_loop(
            0, _NV, scan_body, (neg1, zeros, zeros, zeros, zeros))

        lm = bv[0]
        li = bif[0]
        lz = bz[0]
        ly = by[0]
        lx = bx[0]
        for l in range(1, 16):
            v = bv[l]
            i_ = bif[l]
            btr = (v > lm) | ((v == lm) & (i_ < li))
            lm = jnp.where(btr, v, lm)
            li = jnp.where(btr, i_, li)
            lz = jnp.where(btr, bz[l], lz)
            ly = jnp.where(btr, by[l], ly)
            lx = jnp.where(btr, bx[l], lx)
        pr = jnp.where(lanes == 0, lm, 0.0)
        pr = jnp.where(lanes == 1, li, pr)
        pr = jnp.where(lanes == 2, lz, pr)
        pr = jnp.where(lanes == 3, ly, pr)
        pr = jnp.where(lanes == 4, lx, pr)
        stage[pl.ds(0, 16)] = pr
        even = lax.rem(k, 2) == 0

        @pl.when(even)
        def _pub0():
            pltpu.sync_copy(stage.at[pl.ds(0, 16)], pub.at[pl.ds(sid * 16, 16)])

        @pl.when(jnp.logical_not(even))
        def _pub1():
            pltpu.sync_copy(stage.at[pl.ds(0, 16)],
                            pub.at[pl.ds(256 + sid * 16, 16)])

        plsc.subcore_barrier()

        @pl.when(even)
        def _rd0():
            pltpu.sync_copy(pub.at[pl.ds(0, 256)], rows)

        @pl.when(jnp.logical_not(even))
        def _rd1():
            pltpu.sync_copy(pub.at[pl.ds(256, 256)], rows)

        win = rows[pl.ds(0, 16)]
        for t in range(1, _NT):
            cand = rows[pl.ds(t * 16, 16)]
            va = win[0]
            vb = cand[0]
            btr = (vb > va) | ((vb == va) & (cand[1] < win[1]))
            win = jnp.where(btr, cand, win)
        nm = win[0]
        nz = win[2]
        ny = win[3]
        nx = win[4]
        valid = nm > 0.0

        lane_eq = lanes == lax.rem(k, 16)
        sacc = jnp.where(lane_eq, jnp.where(valid, nm, -1.0), sacc)
        zacc = jnp.where(lane_eq, jnp.where(valid, nz * 0.2, 0.0), zacc)
        yacc = jnp.where(lane_eq, jnp.where(valid, ny, 0.0), yacc)
        xacc = jnp.where(lane_eq, jnp.where(valid, nx, 0.0), xacc)
        cacc = jnp.where(lane_eq, jnp.where(valid, 0, -1), cacc)

        @pl.when((sid == 0) & (lax.rem(k, 16) == 15))
        def _flush():
            o = (k // 16) * 16
            os_v[pl.ds(o, 16)] = sacc
            oz_v[pl.ds(o, 16)] = zacc
            oy_v[pl.ds(o, 16)] = yacc
            ox_v[pl.ds(o, 16)] = xacc
            oc_v[pl.ds(o, 16)] = cacc

        return (nz, ny, nx, sacc, zacc, yacc, xacc, cacc)

    lax.fori_loop(0, _MAX_OUT, body,
                  (jnp.float32(1e9), jnp.float32(1e9), jnp.float32(1e9),
                   neg1, zeros, zeros, zeros, zeroi))

    @pl.when(sid == 0)
    def _out():
        pltpu.sync_copy(os_v, os_h)
        pltpu.sync_copy(oz_v, oz_h)
        pltpu.sync_copy(oy_v, oy_h)
        pltpu.sync_copy(ox_v, ox_h)
        pltpu.sync_copy(oc_v, oc_h)


_nms = pl.kernel(
    _nms_body,
    out_type=(
        jax.ShapeDtypeStruct((_MAX_OUT,), jnp.float32),
        jax.ShapeDtypeStruct((_MAX_OUT,), jnp.float32),
        jax.ShapeDtypeStruct((_MAX_OUT,), jnp.float32),
        jax.ShapeDtypeStruct((_MAX_OUT,), jnp.float32),
        jax.ShapeDtypeStruct((_MAX_OUT,), jnp.int32),
    ),
    mesh=plsc.VectorSubcoreMesh(core_axis_name="c", subcore_axis_name="s", num_cores=1),
    scratch_types=[
        pltpu.VMEM((_PER,), jnp.float32),
        pltpu.VMEM((_PER,), jnp.float32),
        pltpu.VMEM((_PER,), jnp.float32),
        pltpu.VMEM((_PER,), jnp.float32),
        pltpu.VMEM((16,), jnp.float32),
        pltpu.VMEM((_NT * 16,), jnp.float32),
        pltpu.VMEM((_MAX_OUT,), jnp.float32),
        pltpu.VMEM((_MAX_OUT,), jnp.float32),
        pltpu.VMEM((_MAX_OUT,), jnp.float32),
        pltpu.VMEM((_MAX_OUT,), jnp.float32),
        pltpu.VMEM((_MAX_OUT,), jnp.int32),
        pltpu.VMEM_SHARED((2 * _NT * 16,), jnp.float32),
    ],
)


def kernel(cls_logits_0, regressions_0, cls_logits_1, regressions_1,
           cls_logits_2, regressions_2):
    cls = (cls_logits_0, cls_logits_1, cls_logits_2)
    reg = (regressions_0, regressions_1, regressions_2)
    cat = lambda ch, arrs: jnp.concatenate(
        [a[..., ch].reshape(-1) for a in arrs]).reshape(_ROWS, 128)
    l0 = cat(0, cls)
    l1 = cat(1, cls)
    rz = cat(0, reg)
    ry = cat(1, reg)
    rx = cat(2, reg)
    cur, z, y, x = _preprocess(l0, l1, rz, ry, rx,
                               jnp.asarray(_YB), jnp.asarray(_XB),
                               jnp.asarray(_UY), jnp.asarray(_UX),
                               jnp.asarray(_SC))
    out_s, out_z, out_y, out_x, out_c = _nms(
        cur.reshape(-1), z.reshape(-1), y.reshape(-1), x.reshape(-1))
    out_locs = jnp.stack([out_z, out_y, out_x], axis=-1)
    return out_s, out_locs, out_c


# speedup vs baseline: 1.9814x; 1.2187x over previous
import jax
import jax.numpy as jnp
import numpy as np
from jax import lax
from jax.experimental import pallas as pl
from jax.experimental.pallas import tpu as pltpu
from jax.experimental.pallas import tpu_sc as plsc

_LEVELS = ((128, 128, 4.0), (64, 64, 8.0), (32, 32, 16.0))
_N = 21504
_ROWS = _N // 128
_NT = 16
_PER = _N // _NT
_NV = _PER // 16
_MAX_OUT = 512


def _static_arrays():
    ybs, xbs, uys, uxs, scs = [], [], [], [], []
    for h, w, s in _LEVELS:
        gy, gx = np.meshgrid(np.arange(h), np.arange(w), indexing="ij")
        ybs.append((gy + 0.5).astype(np.float32).ravel())
        xbs.append((gx + 0.5).astype(np.float32).ravel())
        uys.append(np.full(h * w, h, np.float32))
        uxs.append(np.full(h * w, w, np.float32))
        scs.append(np.full(h * w, s, np.float32))
    cat = lambda parts: np.concatenate(parts).reshape(_ROWS, 128)
    return cat(ybs), cat(xbs), cat(uys), cat(uxs), cat(scs)


_YB, _XB, _UY, _UX, _SC = _static_arrays()


def _pre_body(l0, l1, rz, ry, rx, yb, xb, uy, ux, sc, cur_o, z_o, y_o, x_o):
    a = l0[...]
    b = l1[...]
    mx = jnp.maximum(a, b)
    e0 = jnp.exp(a - mx)
    e1 = jnp.exp(b - mx)
    s = e0 / (e0 + e1)
    vz = 0.5 + rz[...]
    vy = yb[...] + ry[...]
    vx = xb[...] + rx[...]
    valid = (vz > 0.0) & (vz < 1.0) & (vy > 0.0) & (vy < uy[...]) & (vx > 0.0) & (vx < ux[...])
    cur_o[...] = jnp.where(valid & (s > 0.2), s, -1.0)
    z_o[...] = vz * 5.0
    y_o[...] = vy * sc[...]
    x_o[...] = vx * sc[...]


_preprocess = pl.pallas_call(
    _pre_body,
    out_shape=tuple(jax.ShapeDtypeStruct((_ROWS, 128), jnp.float32) for _ in range(4)),
)


def _better(v, i, V, I):
    return (v > V) | ((v == V) & (i < I))


def _sel5(c, a, b):
    return tuple(jnp.where(c, x, y) for x, y in zip(a, b))


def _bo5(a, b):
    return _sel5(_better(a[0], a[1], b[0], b[1]), a, b)


def _nms_body(cur_h, z_h, y_h, x_h, os_h, oz_h, oy_h, ox_h, oc_h,
              ac, az, ay, ax, stage, rows,
              os_v, oz_v, oy_v, ox_v, oc_v, smi, smf, pub):
    sid = lax.axis_index("s")
    base = sid * _PER
    pltpu.sync_copy(cur_h.at[pl.ds(base, _PER)], ac)
    pltpu.sync_copy(z_h.at[pl.ds(base, _PER)], az)
    pltpu.sync_copy(y_h.at[pl.ds(base, _PER)], ay)
    pltpu.sync_copy(x_h.at[pl.ds(base, _PER)], ax)

    lanes = jnp.arange(16, dtype=jnp.int32)
    lanesf = lanes.astype(jnp.float32)
    basef = (sid * _PER).astype(jnp.float32)
    neg1 = jnp.full((16,), -1.0, jnp.float32)
    zeros = jnp.zeros((16,), jnp.float32)
    negi = jnp.full((16,), -1, jnp.int32)
    far = jnp.float32(1e9)

    for i in range(_MAX_OUT // 16):
        os_v[pl.ds(i * 16, 16)] = neg1
        oz_v[pl.ds(i * 16, 16)] = zeros
        oy_v[pl.ds(i * 16, 16)] = zeros
        ox_v[pl.ds(i * 16, 16)] = zeros
        oc_v[pl.ds(i * 16, 16)] = negi
    smi[0] = jnp.int32(0)
    for j in range(6):
        smf[j] = far

    def body(k, dummy):
        p0 = smi[0]

        @pl.when(p0 < _MAX_OUT)
        def _round():
            w1z = smf[0]
            w1y = smf[1]
            w1x = smf[2]
            w2z = smf[3]
            w2y = smf[4]
            w2x = smf[5]

            def scan_body(i, c):
                b1v, b1i, b1z, b1y, b1x, b2v, b2i, b2z, b2y, b2x = c
                off = i * 16
                cv = ac[pl.ds(off, 16)]
                zz = az[pl.ds(off, 16)]
                yy = ay[pl.ds(off, 16)]
                xx = ax[pl.ds(off, 16)]
                da = zz - w1z
                db = yy - w1y
                dc = xx - w1x
                d2a = da * da + db * db + dc * dc
                ea = zz - w2z
                eb = yy - w2y
                ec = xx - w2x
                d2b = ea * ea + eb * eb + ec * ec
                nc = jnp.where(jnp.minimum(d2a, d2b) < 64.0, -1.0, cv)
                ac[pl.ds(off, 16)] = nc
                fi = basef + off.astype(jnp.float32) + lanesf
                g1 = nc > b1v
                g2 = nc > b2v
                b2v = jnp.where(g1, b1v, jnp.where(g2, nc, b2v))
                b2i = jnp.where(g1, b1i, jnp.where(g2, fi, b2i))
                b2z = jnp.where(g1, b1z, jnp.where(g2, zz, b2z))
                b2y = jnp.where(g1, b1y, jnp.where(g2, yy, b2y))
                b2x = jnp.where(g1, b1x, jnp.where(g2, xx, b2x))
                b1v = jnp.where(g1, nc, b1v)
                b1i = jnp.where(g1, fi, b1i)
                b1z = jnp.where(g1, zz, b1z)
                b1y = jnp.where(g1, yy, b1y)
                b1x = jnp.where(g1, xx, b1x)
                return (b1v, b1i, b1z, b1y, b1x, b2v, b2i, b2z, b2y, b2x)

            sc0 = (neg1, zeros, zeros, zeros, zeros,
                   neg1, zeros, zeros, zeros, zeros)
            b1v, b1i, b1z, b1y, b1x, b2v, b2i, b2z, b2y, b2x = lax.fori_loop(
                0, _NV, scan_body, sc0)

            e1 = [(b1v[l], b1i[l], b1z[l], b1y[l], b1x[l]) for l in range(16)]
            e2 = [(b2v[l], b2i[l], b2z[l], b2y[l], b2x[l]) for l in range(16)]
            T1 = e1[0]
            S = e2[0]
            R = (jnp.float32(-1.0), jnp.float32(0.0), jnp.float32(0.0),
                 jnp.float32(0.0), jnp.float32(0.0))
            for l in range(1, 16):
                btr = _better(e1[l][0], e1[l][1], T1[0], T1[1])
                loser = _sel5(btr, T1, e1[l])
                T1 = _sel5(btr, e1[l], T1)
                S = _sel5(btr, e2[l], S)
                R = _bo5(R, loser)
            T2 = _bo5(R, S)

            pr = jnp.where(lanes == 0, T1[0], 0.0)
            pr = jnp.where(lanes == 1, T1[1], pr)
            pr = jnp.where(lanes == 2, T1[2], pr)
            pr = jnp.where(lanes == 3, T1[3], pr)
            pr = jnp.where(lanes == 4, T1[4], pr)
            pr = jnp.where(lanes == 5, T2[0], pr)
            pr = jnp.where(lanes == 6, T2[1], pr)
            pr = jnp.where(lanes == 7, T2[2], pr)
            pr = jnp.where(lanes == 8, T2[3], pr)
            pr = jnp.where(lanes == 9, T2[4], pr)
            stage[pl.ds(0, 16)] = pr
            even = lax.rem(k, 2) == 0

            @pl.when(even)
            def _pub0():
                pltpu.sync_copy(stage.at[pl.ds(0, 16)],
                                pub.at[pl.ds(sid * 16, 16)])

            @pl.when(jnp.logical_not(even))
            def _pub1():
                pltpu.sync_copy(stage.at[pl.ds(0, 16)],
                                pub.at[pl.ds(256 + sid * 16, 16)])

            plsc.subcore_barrier()

            @pl.when(even)
            def _rd0():
                pltpu.sync_copy(pub.at[pl.ds(0, 256)], rows)

            @pl.when(jnp.logical_not(even))
            def _rd1():
                pltpu.sync_copy(pub.at[pl.ds(256, 256)], rows)

            r0 = rows[pl.ds(0, 16)]
            G1 = (r0[0], r0[1], r0[2], r0[3], r0[4])
            G2 = (r0[5], r0[6], r0[7], r0[8], r0[9])
            for t in range(1, _NT):
                rt = rows[pl.ds(t * 16, 16)]
                a1 = (rt[0], rt[1], rt[2], rt[3], rt[4])
                a2 = (rt[5], rt[6], rt[7], rt[8], rt[9])
                btr = _better(a1[0], a1[1], G1[0], G1[1])
                G2 = _sel5(btr, _bo5(G1, a2), _bo5(G2, a1))
                G1 = _sel5(btr, a1, G1)

            p1ok = G1[0] > 0.0
            dgz = G1[2] - G2[2]
            dgy = G1[3] - G2[3]
            dgx = G1[4] - G2[4]
            dg = dgz * dgz + dgy * dgy + dgx * dgx
            p2ok = p1ok & (G2[0] > 0.0) & jnp.logical_not(dg < 64.0)

            def emit(q, tup):
                blk = (q // 16) * 16
                put = lanes == lax.rem(q, 16)
                os_v[pl.ds(blk, 16)] = jnp.where(put, tup[0], os_v[pl.ds(blk, 16)])
                oz_v[pl.ds(blk, 16)] = jnp.where(put, tup[2] * 0.2, oz_v[pl.ds(blk, 16)])
                oy_v[pl.ds(blk, 16)] = jnp.where(put, tup[3], oy_v[pl.ds(blk, 16)])
                ox_v[pl.ds(blk, 16)] = jnp.where(put, tup[4], ox_v[pl.ds(blk, 16)])
                oc_v[pl.ds(blk, 16)] = jnp.where(put, 0, oc_v[pl.ds(blk, 16)])

            @pl.when(p1ok)
            def _e1():
                emit(p0, G1)

            @pl.when(p2ok & (p0 + 1 < _MAX_OUT))
            def _e2():
                emit(p0 + 1, G2)

            npick = jnp.where(p1ok, jnp.where(p2ok, 2, 1), 0)
            pn = p0 + npick
            smi[0] = jnp.where(pn < _MAX_OUT, pn, jnp.int32(_MAX_OUT))
            smf[0] = jnp.where(p1ok, G1[2], far)
            smf[1] = jnp.where(p1ok, G1[3], far)
            smf[2] = jnp.where(p1ok, G1[4], far)
            smf[3] = jnp.where(p2ok, G2[2], far)
            smf[4] = jnp.where(p2ok, G2[3], far)
            smf[5] = jnp.where(p2ok, G2[4], far)

        return dummy

    lax.fori_loop(0, _MAX_OUT, body, jnp.int32(0))

    @pl.when(sid == 0)
    def _out():
        pltpu.sync_copy(os_v, os_h)
        pltpu.sync_copy(oz_v, oz_h)
        pltpu.sync_copy(oy_v, oy_h)
        pltpu.sync_copy(ox_v, ox_h)
        pltpu.sync_copy(oc_v, oc_h)


_nms = pl.kernel(
    _nms_body,
    out_type=(
        jax.ShapeDtypeStruct((_MAX_OUT,), jnp.float32),
        jax.ShapeDtypeStruct((_MAX_OUT,), jnp.float32),
        jax.ShapeDtypeStruct((_MAX_OUT,), jnp.float32),
        jax.ShapeDtypeStruct((_MAX_OUT,), jnp.float32),
        jax.ShapeDtypeStruct((_MAX_OUT,), jnp.int32),
    ),
    mesh=plsc.VectorSubcoreMesh(core_axis_name="c", subcore_axis_name="s", num_cores=1),
    scratch_types=[
        pltpu.VMEM((_PER,), jnp.float32),
        pltpu.VMEM((_PER,), jnp.float32),
        pltpu.VMEM((_PER,), jnp.float32),
        pltpu.VMEM((_PER,), jnp.float32),
        pltpu.VMEM((16,), jnp.float32),
        pltpu.VMEM((_NT * 16,), jnp.float32),
        pltpu.VMEM((_MAX_OUT,), jnp.float32),
        pltpu.VMEM((_MAX_OUT,), jnp.float32),
        pltpu.VMEM((_MAX_OUT,), jnp.float32),
        pltpu.VMEM((_MAX_OUT,), jnp.float32),
        pltpu.VMEM((_MAX_OUT,), jnp.int32),
        pltpu.SMEM((8,), jnp.int32),
        pltpu.SMEM((8,), jnp.float32),
        pltpu.VMEM_SHARED((2 * _NT * 16,), jnp.float32),
    ],
)


def kernel(cls_logits_0, regressions_0, cls_logits_1, regressions_1,
           cls_logits_2, regressions_2):
    cls = (cls_logits_0, cls_logits_1, cls_logits_2)
    reg = (regressions_0, regressions_1, regressions_2)
    cat = lambda ch, arrs: jnp.concatenate(
        [a[..., ch].reshape(-1) for a in arrs]).reshape(_ROWS, 128)
    l0 = cat(0, cls)
    l1 = cat(1, cls)
    rz = cat(0, reg)
    ry = cat(1, reg)
    rx = cat(2, reg)
    cur, z, y, x = _preprocess(l0, l1, rz, ry, rx,
                               jnp.asarray(_YB), jnp.asarray(_XB),
                               jnp.asarray(_UY), jnp.asarray(_UX),
                               jnp.asarray(_SC))
    out_s, out_z, out_y, out_x, out_c = _nms(
        cur.reshape(-1), z.reshape(-1), y.reshape(-1), x.reshape(-1))
    out_locs = jnp.stack([out_z, out_y, out_x], axis=-1)
    return out_s, out_locs, out_c
